# baseline (speedup 1.0000x reference)
import functools

import jax
import jax.numpy as jnp
from jax import lax
from jax.experimental import pallas as pl
from jax.experimental.pallas import tpu as pltpu
from jax.experimental.pallas import tpu_sc as plsc

_NUM_GENRE = 25
_NUM_ACTOR = 2000
_NUM_DIRECTOR = 500
_EMB = 32

_NC = 2
_NS = 16
_NW = _NC * _NS


def _sc_item_gather_wide(table_wide, idx):
    B = idx.shape[0]
    b_per_w = B // _NW
    mesh = plsc.VectorSubcoreMesh(core_axis_name="c", subcore_axis_name="s")

    @functools.partial(
        pl.kernel,
        mesh=mesh,
        out_type=jax.ShapeDtypeStruct((B, 128), jnp.float32),
        scratch_types=[
            pltpu.VMEM((b_per_w,), jnp.int32),
            pltpu.VMEM((b_per_w,), jnp.int32),
            pltpu.VMEM((b_per_w, 128), jnp.float32),
            pltpu.SemaphoreType.DMA,
        ],
    )
    def gather_kernel(table_hbm, idx_hbm, out_hbm, idx_v, slab_v, gath_v, sem):
        wid = lax.axis_index("s") * _NC + lax.axis_index("c")
        base = wid * b_per_w
        pltpu.sync_copy(idx_hbm.at[pl.ds(base, b_per_w)], idx_v)
        for j in range(b_per_w // 16):
            slab_v[pl.ds(j * 16, 16)] = lax.rem(idx_v[pl.ds(j * 16, 16)], _G4)
        pltpu.async_copy(table_hbm.at[slab_v], gath_v, sem).wait()
        pltpu.sync_copy(gath_v, out_hbm.at[pl.ds(base, b_per_w), :])

    return gather_kernel(table_wide, idx)


_G4 = 25600


def _slabify_body(t0_ref, t1_ref, t2_ref, t3_ref, o_ref):
    o_ref[...] = jnp.concatenate(
        [t0_ref[...].T, t1_ref[...].T, t2_ref[...].T, t3_ref[...].T], axis=1)


def _slabify(table_t):
    RB = 2560
    grid = (_G4 // RB,)
    nb = _G4 // RB
    _, num_item = table_t.shape
    last_valid = num_item // RB

    def spec(m):
        return pl.BlockSpec(
            (_EMB, RB), lambda i, m=m: (0, jnp.minimum(i + m * nb, last_valid)))

    return pl.pallas_call(
        _slabify_body,
        grid=grid,
        in_specs=[spec(0), spec(1), spec(2), spec(3)],
        out_specs=pl.BlockSpec((RB, 128), lambda i: (i, 0)),
        out_shape=jax.ShapeDtypeStruct((_G4, 128), jnp.float32),
    )(table_t, table_t, table_t, table_t)


_KB = 640


def _merge_body(scw_ref, mo_ref, idx_ref, o_ref):
    scw_t = scw_ref[...].T
    sel = idx_ref[...] // _G4
    mb = scw_t.shape[1]
    item_emb = jnp.zeros((_EMB, mb), jnp.float32)
    for m in range(4):
        item_emb = item_emb + jnp.where(
            sel == m, scw_t[32 * m:32 * (m + 1), :], 0.0)
    o_ref[...] = jnp.concatenate([item_emb, mo_ref[...]], axis=0)


def _merge(sc_wide, main_out, idx2d):
    B = sc_wide.shape[0]
    MB = 1024
    return pl.pallas_call(
        _merge_body,
        grid=(B // MB,),
        in_specs=[
            pl.BlockSpec((MB, 128), lambda i: (i, 0)),
            pl.BlockSpec((4 * _EMB, MB), lambda i: (0, i)),
            pl.BlockSpec((1, MB), lambda i: (0, i)),
        ],
        out_specs=pl.BlockSpec((5 * _EMB, MB), lambda i: (0, i)),
        out_shape=jax.ShapeDtypeStruct((5 * _EMB, B), jnp.float32),
    )(sc_wide, main_out, idx2d)


def _tc_body(nk, x0_ref, x1_ref, x2_ref, x3_ref, wcat_ref, wrate_ref,
             o_ref, acc_ref):
    k = pl.program_id(0)
    xi = x0_ref[...]
    b = xi.shape[1]
    sub = xi.shape[0]
    w = wcat_ref[...]
    yt = jnp.dot(w[:, 0:sub], xi.astype(jnp.bfloat16),
                 preferred_element_type=jnp.float32)
    for j, xr in enumerate((x1_ref, x2_ref, x3_ref), start=1):
        yt = yt + jnp.dot(w[:, j * sub:(j + 1) * sub],
                          xr[...].astype(jnp.bfloat16),
                          preferred_element_type=jnp.float32)

    @pl.when(k == 0)
    def _():
        acc_ref[...] = yt
        rate = xi[1:2, :]
        oh = (lax.broadcasted_iota(jnp.int32, (128, b), 0)
              == rate).astype(jnp.bfloat16)
        o_ref[0:_EMB, :] = jnp.dot(
            wrate_ref[...], oh, preferred_element_type=jnp.float32)

    @pl.when(k > 0)
    def _():
        acc_ref[...] += yt

    @pl.when(k == nk - 1)
    def _():
        acc = acc_ref[...]
        d_g = jnp.where(acc[96:97, :] == 0.0, 1.0, acc[96:97, :])
        d_a = jnp.where(acc[97:98, :] == 0.0, 1.0, acc[97:98, :])
        d_d = jnp.where(acc[98:99, :] == 0.0, 1.0, acc[98:99, :])
        o_ref[_EMB:, :] = jnp.concatenate(
            [acc[0:32, :] / d_g, acc[32:64, :] / d_a, acc[64:96, :] / d_d],
            axis=0)


def _tc_compute(xt, wcat_t, wrate_t):
    F, B = xt.shape
    nk = (F + _KB - 1) // _KB
    sub = _KB // 4

    def xspec(j):
        return pl.BlockSpec((sub, B), lambda k, j=j: (4 * k + j, 0))

    return pl.pallas_call(
        functools.partial(_tc_body, nk),
        grid=(nk,),
        in_specs=[
            xspec(0), xspec(1), xspec(2), xspec(3),
            pl.BlockSpec((128, _KB), lambda k: (0, k)),
            pl.BlockSpec((_EMB, 128), lambda k: (0, 0)),
        ],
        out_specs=pl.BlockSpec((4 * _EMB, B), lambda k: (0, 0)),
        out_shape=jax.ShapeDtypeStruct((4 * _EMB, B), jnp.float32),
        scratch_shapes=[pltpu.VMEM((128, B), jnp.float32)],
    )(xt, xt, xt, xt, wcat_t, wrate_t)


def kernel(item_fea, W_item, W_rate, W_genre, W_actor, W_director):
    B, F = item_fea.shape
    num_item = W_item.shape[0]
    g0 = 2
    a0 = g0 + _NUM_GENRE
    d0 = a0 + _NUM_ACTOR
    nk = (F + _KB - 1) // _KB
    f_pad = nk * _KB
    col = lax.broadcasted_iota(jnp.int32, (1, f_pad), 1)
    in_g = (col >= g0) & (col < a0)
    in_a = (col >= a0) & (col < d0)
    in_d = (col >= d0) & (col < F)
    wcat_t = jnp.concatenate([
        jnp.where(in_g, jnp.pad(W_genre.T, ((0, 0), (g0, f_pad - a0))), 0.0),
        jnp.where(in_a, jnp.pad(W_actor.T, ((0, 0), (a0, f_pad - d0))), 0.0),
        jnp.where(in_d, jnp.pad(W_director.T, ((0, 0), (d0, f_pad - F))), 0.0),
        in_g.astype(jnp.float32),
        in_a.astype(jnp.float32),
        in_d.astype(jnp.float32),
        jnp.zeros((128 - 99, f_pad), jnp.float32),
    ], axis=0).astype(jnp.bfloat16)
    wrate_t = jnp.pad(W_rate.T, ((0, 0), (0, 128 - W_rate.shape[0]))
                      ).astype(jnp.bfloat16)

    xt = item_fea.T
    table_wide = _slabify(W_item.T)
    idx = item_fea[:, 0].astype(jnp.int32)
    sc_wide = _sc_item_gather_wide(table_wide, idx)
    main_out = _tc_compute(xt, wcat_t, wrate_t)
    out_t = _merge(sc_wide, main_out, idx.reshape(1, B))
    return out_t.T


# speedup vs baseline: 1.0605x; 1.0605x over previous
import functools

import jax
import jax.numpy as jnp
from jax import lax
from jax.experimental import pallas as pl
from jax.experimental.pallas import tpu as pltpu
from jax.experimental.pallas import tpu_sc as plsc

_NUM_GENRE = 25
_NUM_ACTOR = 2000
_NUM_DIRECTOR = 500
_EMB = 32

_NC = 2
_NS = 16
_NW = _NC * _NS


def _sc_item_gather_wide(table_wide, idx):
    B = idx.shape[0]
    b_per_w = B // _NW
    mesh = plsc.VectorSubcoreMesh(core_axis_name="c", subcore_axis_name="s")

    @functools.partial(
        pl.kernel,
        mesh=mesh,
        out_type=jax.ShapeDtypeStruct((B, 128), jnp.float32),
        scratch_types=[
            pltpu.VMEM((b_per_w,), jnp.int32),
            pltpu.VMEM((b_per_w,), jnp.int32),
            pltpu.VMEM((b_per_w, 128), jnp.float32),
            pltpu.SemaphoreType.DMA,
        ],
    )
    def gather_kernel(table_hbm, idx_hbm, out_hbm, idx_v, slab_v, gath_v, sem):
        wid = lax.axis_index("s") * _NC + lax.axis_index("c")
        base = wid * b_per_w
        pltpu.sync_copy(idx_hbm.at[pl.ds(base, b_per_w)], idx_v)
        for j in range(b_per_w // 16):
            slab_v[pl.ds(j * 16, 16)] = lax.rem(idx_v[pl.ds(j * 16, 16)], _G4)
        pltpu.async_copy(table_hbm.at[slab_v], gath_v, sem).wait()
        pltpu.sync_copy(gath_v, out_hbm.at[pl.ds(base, b_per_w), :])

    return gather_kernel(table_wide, idx)


_G4 = 25600


def _slabify_body(t0_ref, t1_ref, t2_ref, t3_ref, o_ref):
    o_ref[...] = jnp.concatenate(
        [t0_ref[...].T, t1_ref[...].T, t2_ref[...].T, t3_ref[...].T], axis=1)


def _slabify(table_t):
    RB = 2560
    grid = (_G4 // RB,)
    nb = _G4 // RB
    _, num_item = table_t.shape
    last_valid = num_item // RB

    def spec(m):
        return pl.BlockSpec(
            (_EMB, RB), lambda i, m=m: (0, jnp.minimum(i + m * nb, last_valid)))

    return pl.pallas_call(
        _slabify_body,
        grid=grid,
        in_specs=[spec(0), spec(1), spec(2), spec(3)],
        out_specs=pl.BlockSpec((RB, 128), lambda i: (i, 0)),
        out_shape=jax.ShapeDtypeStruct((_G4, 128), jnp.float32),
    )(table_t, table_t, table_t, table_t)


_KB = 640


def _tc_body(nk, x0_ref, x1_ref, x2_ref, x3_ref, wcat_ref, wrate_ref,
             scw_ref, o_ref, acc_ref):
    k = pl.program_id(0)
    xi = x0_ref[...]
    b = xi.shape[1]
    sub = xi.shape[0]
    w = wcat_ref[...]
    yt = jnp.dot(w[:, 0:sub], xi.astype(jnp.bfloat16),
                 preferred_element_type=jnp.float32)
    for j, xr in enumerate((x1_ref, x2_ref, x3_ref), start=1):
        yt = yt + jnp.dot(w[:, j * sub:(j + 1) * sub],
                          xr[...].astype(jnp.bfloat16),
                          preferred_element_type=jnp.float32)

    @pl.when(k == 0)
    def _():
        acc_ref[...] = yt
        rate = xi[1:2, :]
        oh = (lax.broadcasted_iota(jnp.int32, (128, b), 0)
              == rate).astype(jnp.bfloat16)
        o_ref[_EMB:2 * _EMB, :] = jnp.dot(
            wrate_ref[...], oh, preferred_element_type=jnp.float32)
        scw_t = scw_ref[...].T
        sel = xi[0:1, :] // _G4
        item_emb = jnp.zeros((_EMB, b), jnp.float32)
        for m in range(4):
            item_emb = item_emb + jnp.where(
                sel == m, scw_t[32 * m:32 * (m + 1), :], 0.0)
        o_ref[0:_EMB, :] = item_emb

    @pl.when(k > 0)
    def _():
        acc_ref[...] += yt

    @pl.when(k == nk - 1)
    def _():
        acc = acc_ref[...]
        d_g = jnp.where(acc[96:97, :] == 0.0, 1.0, acc[96:97, :])
        d_a = jnp.where(acc[97:98, :] == 0.0, 1.0, acc[97:98, :])
        d_d = jnp.where(acc[98:99, :] == 0.0, 1.0, acc[98:99, :])
        o_ref[2 * _EMB:, :] = jnp.concatenate(
            [acc[0:32, :] / d_g, acc[32:64, :] / d_a, acc[64:96, :] / d_d],
            axis=0)


def _tc_compute(xt, wcat_t, wrate_t, sc_wide):
    F, B = xt.shape
    nk = (F + _KB - 1) // _KB
    sub = _KB // 4

    def xspec(j):
        return pl.BlockSpec((sub, B), lambda k, j=j: (4 * k + j, 0))

    return pl.pallas_call(
        functools.partial(_tc_body, nk),
        grid=(nk,),
        in_specs=[
            xspec(0), xspec(1), xspec(2), xspec(3),
            pl.BlockSpec((128, _KB), lambda k: (0, k)),
            pl.BlockSpec((_EMB, 128), lambda k: (0, 0)),
            pl.BlockSpec((B, 128), lambda k: (0, 0)),
        ],
        out_specs=pl.BlockSpec((5 * _EMB, B), lambda k: (0, 0)),
        out_shape=jax.ShapeDtypeStruct((5 * _EMB, B), jnp.float32),
        scratch_shapes=[pltpu.VMEM((128, B), jnp.float32)],
    )(xt, xt, xt, xt, wcat_t, wrate_t, sc_wide)


def kernel(item_fea, W_item, W_rate, W_genre, W_actor, W_director):
    B, F = item_fea.shape
    num_item = W_item.shape[0]
    g0 = 2
    a0 = g0 + _NUM_GENRE
    d0 = a0 + _NUM_ACTOR
    nk = (F + _KB - 1) // _KB
    f_pad = nk * _KB
    col = lax.broadcasted_iota(jnp.int32, (1, f_pad), 1)
    in_g = (col >= g0) & (col < a0)
    in_a = (col >= a0) & (col < d0)
    in_d = (col >= d0) & (col < F)
    wcat_t = jnp.concatenate([
        jnp.where(in_g, jnp.pad(W_genre.T, ((0, 0), (g0, f_pad - a0))), 0.0),
        jnp.where(in_a, jnp.pad(W_actor.T, ((0, 0), (a0, f_pad - d0))), 0.0),
        jnp.where(in_d, jnp.pad(W_director.T, ((0, 0), (d0, f_pad - F))), 0.0),
        in_g.astype(jnp.float32),
        in_a.astype(jnp.float32),
        in_d.astype(jnp.float32),
        jnp.zeros((128 - 99, f_pad), jnp.float32),
    ], axis=0).astype(jnp.bfloat16)
    wrate_t = jnp.pad(W_rate.T, ((0, 0), (0, 128 - W_rate.shape[0]))
                      ).astype(jnp.bfloat16)

    xt = item_fea.T
    table_wide = _slabify(W_item.T)
    idx = item_fea[:, 0].astype(jnp.int32)
    sc_wide = _sc_item_gather_wide(table_wide, idx)
    out_t = _tc_compute(xt, wcat_t, wrate_t, sc_wide)
    return out_t.T
